# baseline (speedup 1.0000x reference)
import functools

import jax
import jax.numpy as jnp
from jax import lax
from jax.experimental import pallas as pl
from jax.experimental.pallas import tpu as pltpu
from jax.experimental.pallas import tpu_sc as plsc

_N = 64
_MIN_DEPTH = 2.0
_MAX_DEPTH = 6.0
_BLKB = 8192
_CHUNK = 8192
_LANES = 128


def _points_kernel(o_ref, d_ref, pts_ref):
    step = (_MAX_DEPTH - _MIN_DEPTH) / (_N - 1)
    z = _MIN_DEPTH + step * jax.lax.broadcasted_iota(
        jnp.int32, (1, _N, 1), 1
    ).astype(jnp.float32)
    o = o_ref[...]
    d = d_ref[...]
    pts_ref[...] = o[:, None, :] + d[:, None, :] * z


@jax.jit
def kernel(origins, directions):
    B = origins.shape[0]
    o_t = origins.T
    d_t = directions.T
    pts_t = pl.pallas_call(
        _points_kernel,
        grid=(B // _BLKB,),
        in_specs=[
            pl.BlockSpec((3, _BLKB), lambda i: (0, i)),
            pl.BlockSpec((3, _BLKB), lambda i: (0, i)),
        ],
        out_specs=pl.BlockSpec((3, _N, _BLKB), lambda i: (0, 0, i)),
        out_shape=jax.ShapeDtypeStruct((3, _N, B), jnp.float32),
    )(o_t, d_t)

    step = (_MAX_DEPTH - _MIN_DEPTH) / (_N - 1)
    z = _MIN_DEPTH + step * jnp.arange(_N, dtype=jnp.float32)
    crows = _CHUNK // _LANES
    zrep = jnp.broadcast_to(z[:, None, None], (_N, crows, _LANES))

    info = plsc.get_sparse_core_info()
    nw = info.num_cores * info.num_subcores
    rows_per_w = _N // nw
    n_chunks = B // _CHUNK
    mesh = plsc.VectorSubcoreMesh(core_axis_name="c", subcore_axis_name="s")

    @functools.partial(
        pl.kernel,
        mesh=mesh,
        out_type=jax.ShapeDtypeStruct((_N, B // _LANES, _LANES), jnp.float32),
        scratch_types=[pltpu.VMEM((crows, _LANES), jnp.float32)] * rows_per_w
        + [pltpu.SemaphoreType.DMA],
    )
    def _len_sc(zrep_hbm, out_hbm, *rest):
        bufs, sem = rest[:-1], rest[-1]
        w = lax.axis_index("s") * info.num_cores + lax.axis_index("c")
        for r in range(rows_per_w):
            pltpu.sync_copy(zrep_hbm.at[w * rows_per_w + r], bufs[r])
        copies = []
        for r in range(rows_per_w):
            n = w * rows_per_w + r
            for j in range(n_chunks):
                copies.append(
                    pltpu.async_copy(
                        bufs[r], out_hbm.at[n, pl.ds(j * crows, crows)], sem
                    )
                )
        for c in copies:
            c.wait()

    len_t = _len_sc(zrep)
    pts = jnp.transpose(pts_t, (2, 1, 0))
    lengths = lax.reshape(len_t, (B, _N, 1), dimensions=(1, 2, 0))
    return pts, lengths


# speedup vs baseline: 1.6021x; 1.6021x over previous
import jax
import jax.numpy as jnp
from jax.experimental import pallas as pl

_N = 64
_MIN_DEPTH = 2.0
_MAX_DEPTH = 6.0
_NBLK = 8
_LANES = 128


def _raysample_kernel(o_ref, d_ref, pts_ref, len_ref):
    i = pl.program_id(0)
    step = (_MAX_DEPTH - _MIN_DEPTH) / (_N - 1)
    nidx = i * _NBLK + jax.lax.broadcasted_iota(jnp.int32, (1, _NBLK, 1), 1)
    z = _MIN_DEPTH + step * nidx.astype(jnp.float32)
    o = o_ref[...]
    d = d_ref[...]
    pts_ref[...] = o[:, None, :] + d[:, None, :] * z
    zl = _MIN_DEPTH + step * (
        i * _NBLK + jax.lax.broadcasted_iota(jnp.int32, (_NBLK, 1, 1), 0)
    ).astype(jnp.float32)
    len_ref[...] = jnp.broadcast_to(zl, len_ref.shape)


@jax.jit
def kernel(origins, directions):
    B = origins.shape[0]
    o_t = origins.T
    d_t = directions.T
    pts_t, len_t = pl.pallas_call(
        _raysample_kernel,
        grid=(_N // _NBLK,),
        in_specs=[
            pl.BlockSpec((3, B), lambda i: (0, 0)),
            pl.BlockSpec((3, B), lambda i: (0, 0)),
        ],
        out_specs=[
            pl.BlockSpec((3, _NBLK, B), lambda i: (0, i, 0)),
            pl.BlockSpec((_NBLK, B // _LANES, _LANES), lambda i: (i, 0, 0)),
        ],
        out_shape=[
            jax.ShapeDtypeStruct((3, _N, B), jnp.float32),
            jax.ShapeDtypeStruct((_N, B // _LANES, _LANES), jnp.float32),
        ],
    )(o_t, d_t)
    pts = jnp.transpose(pts_t, (2, 1, 0))
    lengths = jax.lax.reshape(len_t, (B, _N, 1), dimensions=(1, 2, 0))
    return pts, lengths


# speedup vs baseline: 1.7290x; 1.0792x over previous
import jax
import jax.numpy as jnp
from jax.experimental import pallas as pl
from jax.experimental.pallas import tpu as pltpu

_N = 64
_MIN_DEPTH = 2.0
_MAX_DEPTH = 6.0
_BLKB = 8192
_LANES = 128


def _raysample_kernel(o_ref, d_ref, pts_ref, len_ref):
    step = (_MAX_DEPTH - _MIN_DEPTH) / (_N - 1)
    z = _MIN_DEPTH + step * jax.lax.broadcasted_iota(
        jnp.int32, (1, _N, 1), 1
    ).astype(jnp.float32)
    o = o_ref[...]
    d = d_ref[...]
    pts_ref[...] = o[:, None, :] + d[:, None, :] * z
    zl = _MIN_DEPTH + step * jax.lax.broadcasted_iota(
        jnp.int32, (_N, 1, 1), 0
    ).astype(jnp.float32)
    len_ref[...] = jnp.broadcast_to(zl, len_ref.shape)


@jax.jit
def kernel(origins, directions):
    B = origins.shape[0]
    o_t = origins.T
    d_t = directions.T
    pts_t, len_t = pl.pallas_call(
        _raysample_kernel,
        grid=(B // _BLKB,),
        in_specs=[
            pl.BlockSpec((3, _BLKB), lambda i: (0, i)),
            pl.BlockSpec((3, _BLKB), lambda i: (0, i)),
        ],
        out_specs=[
            pl.BlockSpec((3, _N, _BLKB), lambda i: (0, 0, i)),
            pl.BlockSpec((_N, _BLKB // _LANES, _LANES), lambda i: (0, i, 0)),
        ],
        out_shape=[
            jax.ShapeDtypeStruct((3, _N, B), jnp.float32),
            jax.ShapeDtypeStruct((_N, B // _LANES, _LANES), jnp.float32),
        ],
        compiler_params=pltpu.CompilerParams(
            dimension_semantics=("parallel",)
        ),
    )(o_t, d_t)
    pts = jnp.transpose(pts_t, (2, 1, 0))
    lengths = jax.lax.reshape(len_t, (B, _N, 1), dimensions=(1, 2, 0))
    return pts, lengths
